# Initial kernel scaffold; baseline (speedup 1.0000x reference)
import jax
import jax.numpy as jnp
from jax.experimental import pallas as pl


def kernel(GCN_input_drug, GCN_input_protein, GCN_edge, GCN_weight, GAT_input, GAT_edge, idx, W_pre, b_pre, W_gcn, b_gcn, W_gat, att_src, att_dst, b_gat, conv_w, conv_b, W_out, b_out):
    raise NotImplementedError("write your pallas kernel here")


# speedup vs baseline: 1.5341x; 1.5341x over previous
import functools

import jax
import jax.numpy as jnp
from jax import lax
from jax.experimental import pallas as pl
from jax.experimental.pallas import tpu as pltpu

PROTEIN = 1512
N_NODES = 10000


def _mm_kernel(a_ref, b_ref, o_ref):
    o_ref[...] = jnp.dot(a_ref[...], b_ref[...],
                         preferred_element_type=jnp.float32)


def _matmul(a, b, bm=512):
    m, k = a.shape
    _, n = b.shape
    grid = (m + bm - 1) // bm
    return pl.pallas_call(
        _mm_kernel,
        grid=(grid,),
        in_specs=[pl.BlockSpec((bm, k), lambda i: (i, 0)),
                  pl.BlockSpec((k, n), lambda i: (0, 0))],
        out_specs=pl.BlockSpec((bm, n), lambda i: (i, 0)),
        out_shape=jax.ShapeDtypeStruct((m, n), jnp.float32),
    )(a, b)


def _gat_mm_kernel(x_ref, w_ref, asrc_ref, adst_ref, h_ref, as_ref, ad_ref):
    h = jnp.dot(x_ref[...], w_ref[...], preferred_element_type=jnp.float32)
    h_ref[...] = h
    as_ref[...] = jnp.sum(h * asrc_ref[...], axis=1, keepdims=True)
    ad_ref[...] = jnp.sum(h * adst_ref[...], axis=1, keepdims=True)


def _gat_matmul(x, w, att_src, att_dst, bm=1024):
    m, k = x.shape
    _, n = w.shape
    grid = (m + bm - 1) // bm
    h, a_s, a_d = pl.pallas_call(
        _gat_mm_kernel,
        grid=(grid,),
        in_specs=[pl.BlockSpec((bm, k), lambda i: (i, 0)),
                  pl.BlockSpec((k, n), lambda i: (0, 0)),
                  pl.BlockSpec((1, n), lambda i: (0, 0)),
                  pl.BlockSpec((1, n), lambda i: (0, 0))],
        out_specs=[pl.BlockSpec((bm, n), lambda i: (i, 0)),
                   pl.BlockSpec((bm, 1), lambda i: (i, 0)),
                   pl.BlockSpec((bm, 1), lambda i: (i, 0))],
        out_shape=[jax.ShapeDtypeStruct((m, n), jnp.float32),
                   jax.ShapeDtypeStruct((m, 1), jnp.float32),
                   jax.ShapeDtypeStruct((m, 1), jnp.float32)],
    )(x, w, att_src.reshape(1, n), att_dst.reshape(1, n))
    return h, a_s[:, 0], a_d[:, 0]


def _pre_kernel(accg_ref, acca_ref, hg_ref, ha_ref, dinv_ref, den_ref,
                exs_ref, bg_ref, ba_ref, pre_ref, pm_ref, ps_ref):
    dinv = dinv_ref[...]
    exs = exs_ref[...]
    den = den_ref[...] + exs
    gcn = dinv * accg_ref[...] + (dinv * dinv) * hg_ref[...] + bg_ref[...]
    gat = (acca_ref[...] + exs * ha_ref[...]) / den + ba_ref[...]
    pre = jnp.concatenate([gcn, gat], axis=1)
    pre_ref[...] = pre
    m = jnp.max(pre, axis=0, keepdims=True)
    pm_ref[...] = m[None]
    ps_ref[...] = jnp.sum(jnp.exp(pre - m), axis=0, keepdims=True)[None]


def _pre(acc_gcn, acc_gat, h_gcn, h_gat, dinv, denom_e, ex_self,
         b_gcn, b_gat, bn=2000):
    n, f = h_gcn.shape
    grid = n // bn
    return pl.pallas_call(
        _pre_kernel,
        grid=(grid,),
        in_specs=[
            pl.BlockSpec((bn, f), lambda i: (i, 0)),
            pl.BlockSpec((bn, f), lambda i: (i, 0)),
            pl.BlockSpec((bn, f), lambda i: (i, 0)),
            pl.BlockSpec((bn, f), lambda i: (i, 0)),
            pl.BlockSpec((bn, 1), lambda i: (i, 0)),
            pl.BlockSpec((bn, 1), lambda i: (i, 0)),
            pl.BlockSpec((bn, 1), lambda i: (i, 0)),
            pl.BlockSpec((1, f), lambda i: (0, 0)),
            pl.BlockSpec((1, f), lambda i: (0, 0)),
        ],
        out_specs=[pl.BlockSpec((bn, 2 * f), lambda i: (i, 0)),
                   pl.BlockSpec((1, 1, 2 * f), lambda i: (i, 0, 0)),
                   pl.BlockSpec((1, 1, 2 * f), lambda i: (i, 0, 0))],
        out_shape=[jax.ShapeDtypeStruct((n, 2 * f), jnp.float32),
                   jax.ShapeDtypeStruct((grid, 1, 2 * f), jnp.float32),
                   jax.ShapeDtypeStruct((grid, 1, 2 * f), jnp.float32)],
    )(acc_gcn, acc_gat, h_gcn, h_gat, dinv.reshape(n, 1),
      denom_e.reshape(n, 1), ex_self.reshape(n, 1),
      b_gcn.reshape(1, f), b_gat.reshape(1, f))


def _colfix_kernel(pm_ref, ps_ref, off_ref):
    pm = pm_ref[...]
    ps = ps_ref[...]
    m = jnp.max(pm, axis=0, keepdims=True)
    s = jnp.sum(ps * jnp.exp(pm - m), axis=0, keepdims=True)
    off_ref[...] = m + jnp.log(s)


def _colfix(pm, ps):
    g, f2 = pm.shape
    return pl.pallas_call(
        _colfix_kernel,
        in_specs=[pl.BlockSpec((g, f2), lambda: (0, 0)),
                  pl.BlockSpec((g, f2), lambda: (0, 0))],
        out_specs=pl.BlockSpec((1, f2), lambda: (0, 0)),
        out_shape=jax.ShapeDtypeStruct((1, f2), jnp.float32),
    )(pm, ps)


def _head_kernel(fre_ref, fro_ref, fpe_ref, fpo_ref, offe_ref, offo_ref,
                 cw_ref, cb_ref, wo_ref, bo_ref, o_ref):
    b = fre_ref.shape[0]
    hw = fre_ref.shape[1]
    z1 = jnp.zeros((b, 1), jnp.float32)
    offe = offe_ref[...]
    offo = offo_ref[...]

    def padded(x):
        return jnp.concatenate([z1, x, z1], axis=1)

    pads = [(padded(fre_ref[...] - offe), padded(fro_ref[...] - offo)),
            (padded(fpe_ref[...] - offe), padded(fpo_ref[...] - offo))]
    cw = cw_ref[...]
    cb = cb_ref[...]
    wo = wo_ref[...]
    acc0 = jnp.zeros((b, 1), jnp.float32)
    acc1 = jnp.zeros((b, 1), jnp.float32)
    terms = [((2, 0),), ((1, 0), (2, 1)), ((0, 0), (1, 1)), ((0, 1),)]
    for o in range(16):
        rows = []
        for y in range(4):
            for q in range(2):
                r = jnp.zeros((b, hw), jnp.float32)
                for kh, which in terms[y]:
                    ep, op = pads[which]
                    for kw in range(5):
                        u = q + kw - 2
                        if u % 2 == 0:
                            j = u // 2 + 1
                            r = r + ep[:, j:j + hw] * cw[o, kh * 5 + kw]
                        else:
                            j = (u - 1) // 2 + 1
                            r = r + op[:, j:j + hw] * cw[o, kh * 5 + kw]
                r = r + cb[0, o]
                rows.append(jnp.where(r > 0, r, 0.01 * r))
        e0 = jnp.tanh((rows[0] + rows[1] + rows[2] + rows[3]) * 0.25)
        e1 = jnp.tanh((rows[4] + rows[5] + rows[6] + rows[7]) * 0.25)
        base = o * 400
        acc0 += (jnp.dot(e0, wo[base:base + 200, 0:1],
                         preferred_element_type=jnp.float32)
                 + jnp.dot(e1, wo[base + 200:base + 400, 0:1],
                           preferred_element_type=jnp.float32))
        acc1 += (jnp.dot(e0, wo[base:base + 200, 1:2],
                         preferred_element_type=jnp.float32)
                 + jnp.dot(e1, wo[base + 200:base + 400, 1:2],
                           preferred_element_type=jnp.float32))
    o_ref[...] = jnp.concatenate([acc0, acc1], axis=1) + bo_ref[...]


def _head(fr, fp, off, conv_w, conv_b, W_out, b_out, bb=1024):
    b, w = fr.shape
    hw = w // 2
    grid = b // bb
    cw = conv_w.reshape(16, 15)
    fre, fro = fr[:, 0::2], fr[:, 1::2]
    fpe, fpo = fp[:, 0::2], fp[:, 1::2]
    offe, offo = off[:, 0::2], off[:, 1::2]
    return pl.pallas_call(
        _head_kernel,
        grid=(grid,),
        in_specs=[pl.BlockSpec((bb, hw), lambda i: (i, 0)),
                  pl.BlockSpec((bb, hw), lambda i: (i, 0)),
                  pl.BlockSpec((bb, hw), lambda i: (i, 0)),
                  pl.BlockSpec((bb, hw), lambda i: (i, 0)),
                  pl.BlockSpec((1, hw), lambda i: (0, 0)),
                  pl.BlockSpec((1, hw), lambda i: (0, 0)),
                  pl.BlockSpec((16, 15), lambda i: (0, 0)),
                  pl.BlockSpec((1, 16), lambda i: (0, 0)),
                  pl.BlockSpec((6400, 2), lambda i: (0, 0)),
                  pl.BlockSpec((1, 2), lambda i: (0, 0))],
        out_specs=pl.BlockSpec((bb, 2), lambda i: (i, 0)),
        out_shape=jax.ShapeDtypeStruct((b, 2), jnp.float32),
    )(fre, fro, fpe, fpo, offe, offo, cw, conv_b.reshape(1, 16), W_out,
      b_out.reshape(1, 2))


def kernel(GCN_input_drug, GCN_input_protein, GCN_edge, GCN_weight,
           GAT_input, GAT_edge, idx, W_pre, b_pre, W_gcn, b_gcn,
           W_gat, att_src, att_dst, b_gat, conv_w, conv_b, W_out, b_out):
    n = N_NODES
    f = 200

    prot = _matmul(GCN_input_protein, W_pre, bm=512) + b_pre
    h_prot = _matmul(prot, W_gcn, bm=512)
    h_drug = _matmul(GCN_input_drug, W_gcn, bm=1024)
    h_gcn = jnp.concatenate([h_drug, h_prot], axis=0)
    h_gat, a_s, a_d = _gat_matmul(GAT_input, W_gat, att_src, att_dst)
    m = jnp.max(a_s) + jnp.max(a_d)

    src_g = GCN_edge[:, 0]
    dst_g = GCN_edge[:, 1]
    src_a = GAT_edge[:, 0]
    dst_a = GAT_edge[:, 1]

    deg = jax.ops.segment_sum(GCN_weight, dst_g, num_segments=n) + 1.0
    e_edge = jax.nn.leaky_relu(a_s[src_a] + a_d[dst_a], 0.2)
    ex_edge = jnp.exp(e_edge - m)
    denom_e = jax.ops.segment_sum(ex_edge, dst_a, num_segments=n)

    dinv = lax.rsqrt(deg)
    ex_self = jnp.exp(jax.nn.leaky_relu(a_s + a_d, 0.2) - m)

    scale_g = GCN_weight * dinv[src_g]
    acc_gcn = jax.ops.segment_sum(scale_g[:, None] * h_gcn[src_g], dst_g,
                                  num_segments=n)
    acc_gat = jax.ops.segment_sum(ex_edge[:, None] * h_gat[src_a], dst_a,
                                  num_segments=n)

    pre, pm, ps = _pre(acc_gcn, acc_gat, h_gcn, h_gat, dinv, denom_e,
                       ex_self, b_gcn, b_gat)
    off = _colfix(pm[:, 0, :], ps[:, 0, :])

    r = idx // PROTEIN
    p = idx % PROTEIN
    fr = pre[r]
    fp = pre[p]

    return _head(fr, fp, off, conv_w, conv_b, W_out, b_out)
